# baseline (speedup 1.0000x reference)
import functools

import jax
import jax.numpy as jnp
from jax import lax
from jax.experimental import pallas as pl
from jax.experimental.pallas import tpu as pltpu
from jax.experimental.pallas import tpu_sc as plsc

HASH_BIN = 100000
EMB_DIM = 32
H1 = 32
H2 = 64
BATCH = 16384
FIELDS = 26

TOTAL = BATCH * FIELDS
NW = 32
ROWS_PER_W = TOTAL // NW
CHUNK = 128
NCHUNK = ROWS_PER_W // CHUNK

HALF_ROWS = HASH_BIN // 2
ROW_BLOCK = 2000
N_BLOCKS = HALF_ROWS // ROW_BLOCK


def _mlp_body(t_ref, w1_ref, b1_ref, w2_ref, b2_ref, o_ref):
    x = t_ref[...]
    w1 = w1_ref[...]
    b1 = b1_ref[...]
    w2 = w2_ref[...]
    b2 = b2_ref[...]
    for half in (0, 1):
        xs = x[:, half * EMB_DIM:(half + 1) * EMB_DIM]
        h = jnp.dot(xs, w1, preferred_element_type=jnp.float32)
        h = jnp.maximum(h + b1, 0.0)
        o = jnp.dot(h, w2, preferred_element_type=jnp.float32)
        o = jnp.maximum(o + b2, 0.0).astype(jnp.bfloat16)
        o_ref[:, half * H2:(half + 1) * H2] = o


def _table_mlp(table, W1, b1, W2, b2):
    return pl.pallas_call(
        _mlp_body,
        grid=(N_BLOCKS,),
        in_specs=[
            pl.BlockSpec((ROW_BLOCK, 2 * EMB_DIM), lambda i: (i, 0)),
            pl.BlockSpec((EMB_DIM, H1), lambda i: (0, 0)),
            pl.BlockSpec((1, H1), lambda i: (0, 0)),
            pl.BlockSpec((H1, H2), lambda i: (0, 0)),
            pl.BlockSpec((1, H2), lambda i: (0, 0)),
        ],
        out_specs=pl.BlockSpec((ROW_BLOCK, 2 * H2), lambda i: (i, 0)),
        out_shape=jax.ShapeDtypeStruct((HALF_ROWS, 2 * H2), jnp.bfloat16),
    )(table.reshape(HALF_ROWS, 2 * EMB_DIM), W1, b1, W2, b2)


NSLICE = 2
SLICE_ROWS = TOTAL // NSLICE
SLICE_B = BATCH // NSLICE
RELAYOUT_BB = 32


def _relayout_slice_body(x_ref, o_ref):
    x = x_ref[...].astype(jnp.float32)
    o_ref[...] = x.reshape(RELAYOUT_BB * 8, FIELDS * H2)


def _relayout_slice_buf_body(x_ref, b_ref, o_ref):
    del b_ref
    x = x_ref[...].astype(jnp.float32)
    o_ref[...] = x.reshape(RELAYOUT_BB * 8, FIELDS * H2)


def _relayout_slice(flat_s, buf, s):
    x = flat_s.reshape(SLICE_B // 8, 104, 128)
    blocks = SLICE_B // 8 // RELAYOUT_BB
    off = s * blocks
    x_spec = pl.BlockSpec((RELAYOUT_BB, 104, 128), lambda i: (i, 0, 0))
    o_spec = pl.BlockSpec((RELAYOUT_BB * 8, FIELDS * H2),
                          lambda i, off=off: (i + off, 0))
    o_shape = jax.ShapeDtypeStruct((BATCH, FIELDS * H2), jnp.float32)
    if buf is None:
        return pl.pallas_call(
            _relayout_slice_body, grid=(blocks,), in_specs=[x_spec],
            out_specs=o_spec, out_shape=o_shape,
        )(x)
    return pl.pallas_call(
        _relayout_slice_buf_body, grid=(blocks,),
        in_specs=[x_spec, pl.BlockSpec(memory_space=pl.ANY)],
        out_specs=o_spec, out_shape=o_shape,
        input_output_aliases={1: 0},
    )(x, buf)


@functools.lru_cache(maxsize=None)
def _make_sc_gather(nrows):
    rows_per_w = nrows // NW
    nchunk = rows_per_w // CHUNK
    mesh = plsc.VectorSubcoreMesh(core_axis_name="c", subcore_axis_name="s")

    @functools.partial(
        pl.kernel,
        out_type=jax.ShapeDtypeStruct((nrows, H2), jnp.bfloat16),
        mesh=mesh,
        scratch_types=[
            pltpu.VMEM((nchunk, CHUNK), jnp.int32),
            pltpu.VMEM((CHUNK, H2), jnp.bfloat16),
            pltpu.VMEM((CHUNK, H2), jnp.bfloat16),
            pltpu.VMEM((CHUNK, H2), jnp.bfloat16),
            pltpu.VMEM((CHUNK, H2), jnp.bfloat16),
            pltpu.SemaphoreType.DMA,
            pltpu.SemaphoreType.DMA,
            pltpu.SemaphoreType.DMA,
            pltpu.SemaphoreType.DMA,
            pltpu.SemaphoreType.DMA,
            pltpu.SemaphoreType.DMA,
            pltpu.SemaphoreType.DMA,
            pltpu.SemaphoreType.DMA,
        ],
        compiler_params=pltpu.CompilerParams(use_tc_tiling_on_sc=False),
    )
    def _sc_gather(t2_hbm, idx_hbm, out_hbm, idx_v, rows0, rows1, rows2,
                   rows3, gsem0, gsem1, gsem2, gsem3, wsem0, wsem1, wsem2,
                   wsem3):
        wid = lax.axis_index("s") * 2 + lax.axis_index("c")
        pltpu.sync_copy(idx_hbm.at[wid], idx_v)
        base = wid * rows_per_w
        rows = (rows0, rows1, rows2, rows3)
        gsem = (gsem0, gsem1, gsem2, gsem3)
        wsem = (wsem0, wsem1, wsem2, wsem3)

        def g_start(j, b):
            pltpu.async_copy(t2_hbm.at[idx_v.at[j]], rows[b], gsem[b])

        def g_wait(b):
            pltpu.make_async_copy(t2_hbm.at[idx_v.at[0]], rows[b],
                                  gsem[b]).wait()

        def w_start(j, b):
            pltpu.async_copy(rows[b],
                             out_hbm.at[pl.ds(base + j * CHUNK, CHUNK)],
                             wsem[b])

        def w_wait(b):
            pltpu.make_async_copy(rows[b], out_hbm.at[pl.ds(base, CHUNK)],
                                  wsem[b]).wait()

        g_start(0, 0)
        g_start(1, 1)
        g_start(2, 2)

        def step(j, u):
            b = u
            bp = (u + 3) % 4

            @pl.when(j >= 1)
            def _():
                w_wait(bp)

            @pl.when(j + 3 < nchunk)
            def _():
                g_start(j + 3, bp)

            g_wait(b)
            w_start(j, b)

        def body(i, carry):
            for u in range(4):
                step(4 * i + u, u)
            return carry

        lax.fori_loop(0, nchunk // 4, body, 0)
        w_wait((nchunk - 1) % 4)

    return _sc_gather


def kernel(inputs, table, W1, b1, W2, b2):
    t2 = _table_mlp(table, W1.astype(jnp.float32), b1.reshape(1, H1),
                    W2.astype(jnp.float32), b2.reshape(1, H2))
    t2 = t2.reshape(HASH_BIN, H2)
    nchunk_s = SLICE_ROWS // NW // CHUNK
    idx = jnp.mod(inputs, HASH_BIN).reshape(NSLICE, NW, nchunk_s, CHUNK)
    gather = _make_sc_gather(SLICE_ROWS)
    buf = None
    for s in range(NSLICE):
        out_s = gather(t2, idx[s])
        buf = _relayout_slice(out_s, buf, s)
    return buf


# speedup vs baseline: 1.9199x; 1.9199x over previous
import functools

import jax
import jax.numpy as jnp
from jax import lax
from jax.experimental import pallas as pl
from jax.experimental.pallas import tpu as pltpu
from jax.experimental.pallas import tpu_sc as plsc

HASH_BIN = 100000
EMB_DIM = 32
H1 = 32
H2 = 64
BATCH = 16384
FIELDS = 26

TOTAL = BATCH * FIELDS
NW = 32
ROWS_PER_W = TOTAL // NW
CHUNK = 128
NCHUNK = ROWS_PER_W // CHUNK

HALF_ROWS = HASH_BIN // 2
ROW_BLOCK = 2000
N_BLOCKS = HALF_ROWS // ROW_BLOCK


def _mlp_body(ta_ref, tb_ref, w1_ref, b1_ref, w2_ref, b2_ref, o_ref):
    w1 = w1_ref[...]
    b1 = b1_ref[...]
    w2 = w2_ref[...]
    b2 = b2_ref[...]
    for half, t_ref in ((0, ta_ref), (1, tb_ref)):
        h = jnp.dot(t_ref[...], w1, preferred_element_type=jnp.float32)
        h = jnp.maximum(h + b1, 0.0)
        o = jnp.dot(h, w2, preferred_element_type=jnp.float32)
        o_ref[:, half * H2:(half + 1) * H2] = jnp.maximum(o + b2, 0.0)


def _table_mlp(table, W1, b1, W2, b2):
    return pl.pallas_call(
        _mlp_body,
        grid=(N_BLOCKS,),
        in_specs=[
            pl.BlockSpec((ROW_BLOCK, EMB_DIM), lambda i: (i, 0)),
            pl.BlockSpec((ROW_BLOCK, EMB_DIM),
                         lambda i: (i + N_BLOCKS, 0)),
            pl.BlockSpec((EMB_DIM, H1), lambda i: (0, 0)),
            pl.BlockSpec((1, H1), lambda i: (0, 0)),
            pl.BlockSpec((H1, H2), lambda i: (0, 0)),
            pl.BlockSpec((1, H2), lambda i: (0, 0)),
        ],
        out_specs=pl.BlockSpec((ROW_BLOCK, 2 * H2), lambda i: (i, 0)),
        out_shape=jax.ShapeDtypeStruct((HALF_ROWS, 2 * H2), jnp.float32),
    )(table, table, W1, b1, W2, b2)


NSLICE = 2
SLICE_ROWS = TOTAL // NSLICE
SLICE_B = BATCH // NSLICE
RELAYOUT_BB = 32


def _relayout_slice_body(x_ref, o_ref):
    o_ref[...] = x_ref[...].reshape(RELAYOUT_BB * 8, FIELDS * H2)


def _relayout_slice_buf_body(x_ref, b_ref, o_ref):
    del b_ref
    o_ref[...] = x_ref[...].reshape(RELAYOUT_BB * 8, FIELDS * H2)


def _relayout_slice(flat_s, buf, s):
    x = flat_s.reshape(SLICE_B // 8, 104, 128)
    blocks = SLICE_B // 8 // RELAYOUT_BB
    off = s * blocks
    x_spec = pl.BlockSpec((RELAYOUT_BB, 104, 128), lambda i: (i, 0, 0))
    o_spec = pl.BlockSpec((RELAYOUT_BB * 8, FIELDS * H2),
                          lambda i, off=off: (i + off, 0))
    o_shape = jax.ShapeDtypeStruct((BATCH, FIELDS * H2), jnp.float32)
    if buf is None:
        return pl.pallas_call(
            _relayout_slice_body, grid=(blocks,), in_specs=[x_spec],
            out_specs=o_spec, out_shape=o_shape,
        )(x)
    return pl.pallas_call(
        _relayout_slice_buf_body, grid=(blocks,),
        in_specs=[x_spec, pl.BlockSpec(memory_space=pl.ANY)],
        out_specs=o_spec, out_shape=o_shape,
        input_output_aliases={1: 0},
    )(x, buf)


@functools.lru_cache(maxsize=None)
def _make_sc_gather(nrows):
    rows_per_w = nrows // NW
    nchunk = rows_per_w // CHUNK
    mesh = plsc.VectorSubcoreMesh(core_axis_name="c", subcore_axis_name="s")

    @functools.partial(
        pl.kernel,
        out_type=jax.ShapeDtypeStruct((nrows, H2), jnp.float32),
        mesh=mesh,
        scratch_types=[
            pltpu.VMEM((nchunk, CHUNK), jnp.int32),
            pltpu.VMEM((CHUNK, H2), jnp.float32),
            pltpu.VMEM((CHUNK, H2), jnp.float32),
            pltpu.VMEM((CHUNK, H2), jnp.float32),
            pltpu.VMEM((CHUNK, H2), jnp.float32),
            pltpu.SemaphoreType.DMA,
            pltpu.SemaphoreType.DMA,
            pltpu.SemaphoreType.DMA,
            pltpu.SemaphoreType.DMA,
            pltpu.SemaphoreType.DMA,
            pltpu.SemaphoreType.DMA,
            pltpu.SemaphoreType.DMA,
            pltpu.SemaphoreType.DMA,
        ],
        compiler_params=pltpu.CompilerParams(use_tc_tiling_on_sc=False),
    )
    def _sc_gather(t2_hbm, idx_hbm, out_hbm, idx_v, rows0, rows1, rows2,
                   rows3, gsem0, gsem1, gsem2, gsem3, wsem0, wsem1, wsem2,
                   wsem3):
        wid = lax.axis_index("s") * 2 + lax.axis_index("c")
        pltpu.sync_copy(idx_hbm.at[wid], idx_v)
        base = wid * rows_per_w
        rows = (rows0, rows1, rows2, rows3)
        gsem = (gsem0, gsem1, gsem2, gsem3)
        wsem = (wsem0, wsem1, wsem2, wsem3)

        def g_start(j, b):
            pltpu.async_copy(t2_hbm.at[idx_v.at[j]], rows[b], gsem[b])

        def g_wait(b):
            pltpu.make_async_copy(t2_hbm.at[idx_v.at[0]], rows[b],
                                  gsem[b]).wait()

        def w_start(j, b):
            pltpu.async_copy(rows[b],
                             out_hbm.at[pl.ds(base + j * CHUNK, CHUNK)],
                             wsem[b])

        def w_wait(b):
            pltpu.make_async_copy(rows[b], out_hbm.at[pl.ds(base, CHUNK)],
                                  wsem[b]).wait()

        g_start(0, 0)
        g_start(1, 1)
        g_start(2, 2)

        def step(j, u):
            b = u
            bp = (u + 3) % 4

            @pl.when(j >= 1)
            def _():
                w_wait(bp)

            @pl.when(j + 3 < nchunk)
            def _():
                g_start(j + 3, bp)

            g_wait(b)
            w_start(j, b)

        def body(i, carry):
            for u in range(4):
                step(4 * i + u, u)
            return carry

        lax.fori_loop(0, nchunk // 4, body, 0)
        w_wait((nchunk - 1) % 4)

    return _sc_gather


def kernel(inputs, table, W1, b1, W2, b2):
    t2 = _table_mlp(table, W1.astype(jnp.float32), b1.reshape(1, H1),
                    W2.astype(jnp.float32), b2.reshape(1, H2))
    t2 = t2.reshape(HASH_BIN, H2)
    nchunk_s = SLICE_ROWS // NW // CHUNK
    v = jnp.mod(inputs, HASH_BIN)
    v = 2 * jnp.mod(v, HALF_ROWS) + v // HALF_ROWS
    idx = v.reshape(NSLICE, NW, nchunk_s, CHUNK)
    gather = _make_sc_gather(SLICE_ROWS)
    buf = None
    for s in range(NSLICE):
        out_s = gather(t2, idx[s])
        buf = _relayout_slice(out_s, buf, s)
    return buf
